# Initial kernel scaffold; baseline (speedup 1.0000x reference)
import jax
import jax.numpy as jnp
from jax.experimental import pallas as pl


def kernel(features, edge_index, W1, b1, Wc1, bc1, Wc2, bc2, W2, b2):
    raise NotImplementedError("write your pallas kernel here")


# speedup vs baseline: 8.6400x; 8.6400x over previous
import functools

import jax
import jax.numpy as jnp
from jax import lax
from jax.experimental import pallas as pl
from jax.experimental.pallas import tpu as pltpu
from jax.experimental.pallas import tpu_sc as plsc

_N = 10000
_E = 320000
_NC = 2
_NS = 16
_EPW = _E // (_NC * _NS)
_RPT = 624
_TAIL = _N - _NS * _RPT


@functools.partial(jax.jit, static_argnames=("d", "chunk"))
def _sc_aggregate(h, src, dst, zeros, *, d, chunk):
    n_chunks = _EPW // chunk
    mesh = plsc.VectorSubcoreMesh(core_axis_name="c", subcore_axis_name="s",
                                  num_cores=_NC, num_subcores=_NS)

    @functools.partial(
        pl.kernel,
        out_type=jax.ShapeDtypeStruct((_NC, _N, d), jnp.float32),
        mesh=mesh,
        compiler_params=pltpu.CompilerParams(use_tc_tiling_on_sc=False),
        scratch_types=[
            pltpu.VMEM((chunk,), jnp.int32),
            pltpu.VMEM((chunk,), jnp.int32),
            pltpu.VMEM((chunk, d), jnp.float32),
            pltpu.VMEM_SHARED((_N, d), jnp.float32),
            pltpu.SemaphoreType.DMA,
        ],
    )
    def agg(h_hbm, src_hbm, dst_hbm, z_hbm, out_hbm, src_v, dst_v, rows_v,
            acc_sh, sem):
        c = lax.axis_index("c")
        s = lax.axis_index("s")
        base = (c * _NS + s) * _EPW
        pltpu.sync_copy(z_hbm.at[pl.ds(s * _RPT, _RPT)],
                        acc_sh.at[pl.ds(s * _RPT, _RPT)])

        @pl.when(s == 0)
        def _():
            pltpu.sync_copy(z_hbm.at[pl.ds(_NS * _RPT, _TAIL)],
                            acc_sh.at[pl.ds(_NS * _RPT, _TAIL)])

        plsc.subcore_barrier()

        def body(i, carry):
            off = base + i * chunk
            pltpu.sync_copy(src_hbm.at[pl.ds(off, chunk)], src_v)
            pltpu.sync_copy(dst_hbm.at[pl.ds(off, chunk)], dst_v)
            pltpu.async_copy(h_hbm.at[src_v], rows_v, sem).wait()
            pltpu.sync_copy(rows_v, acc_sh.at[dst_v], add=True)
            return carry

        lax.fori_loop(0, n_chunks, body, 0)
        plsc.subcore_barrier()
        pltpu.sync_copy(acc_sh.at[pl.ds(s * _RPT, _RPT)],
                        out_hbm.at[c, pl.ds(s * _RPT, _RPT)])

        @pl.when(s == 0)
        def _():
            pltpu.sync_copy(acc_sh.at[pl.ds(_NS * _RPT, _TAIL)],
                            out_hbm.at[c, pl.ds(_NS * _RPT, _TAIL)])

    return agg(h, src, dst, zeros)


_BLK = 1000


def _row_specs(n_in, shapes_in, shapes_out):
    in_specs = [pl.BlockSpec((_BLK,) + s[1:], lambda i, *, _nd=len(s): (i,) + (0,) * (_nd - 1))
                for s in shapes_in[:n_in]]
    in_specs += [pl.BlockSpec(s, lambda i, *, _nd=len(s): (0,) * _nd)
                 for s in shapes_in[n_in:]]
    out_specs = [pl.BlockSpec((_BLK,) + s[1:], lambda i, *, _nd=len(s): (i,) + (0,) * (_nd - 1))
                 for s in shapes_out]
    return in_specs, out_specs


def _tc_call(body, row_args, full_args, out_shapes):
    shapes_in = [a.shape for a in row_args] + [a.shape for a in full_args]
    in_specs, out_specs = _row_specs(len(row_args), shapes_in, out_shapes)
    return pl.pallas_call(
        body,
        grid=(_N // _BLK,),
        in_specs=in_specs,
        out_specs=out_specs if len(out_shapes) > 1 else out_specs[0],
        out_shape=[jax.ShapeDtypeStruct(s, jnp.float32) for s in out_shapes]
        if len(out_shapes) > 1 else jax.ShapeDtypeStruct(out_shapes[0], jnp.float32),
    )(*row_args, *full_args)


def _stage1_body(f_ref, w1_ref, b1_ref, wc1_ref, x_ref, h_ref):
    x = jnp.maximum(jnp.dot(f_ref[...], w1_ref[...],
                            preferred_element_type=jnp.float32) + b1_ref[...], 0.0)
    x_ref[...] = x
    h_ref[...] = jnp.dot(x, wc1_ref[...], preferred_element_type=jnp.float32)


def _combine_body(p0_ref, p1_ref, b_ref, w_ref, x_ref, h_ref):
    x = p0_ref[...] + p1_ref[...] + b_ref[...]
    x_ref[...] = x
    h_ref[...] = jnp.dot(x, w_ref[...], preferred_element_type=jnp.float32)


def _combine2_body(p0_ref, p1_ref, xa_ref, b_ref, wtop_ref, wbot_ref,
                   x_ref, h_ref):
    x = p0_ref[...] + p1_ref[...] + b_ref[...]
    x_ref[...] = x
    h_ref[...] = (jnp.dot(xa_ref[...], wtop_ref[...], preferred_element_type=jnp.float32)
                  + jnp.dot(x, wbot_ref[...], preferred_element_type=jnp.float32))


def _final_body(p0_ref, p1_ref, x_ref, x11_ref, x12_ref, x21_ref,
                bc2_ref, w2_ref, b2_ref, o_ref):
    x22 = p0_ref[...] + p1_ref[...] + bc2_ref[...]
    w2 = w2_ref[...]
    logits = (jnp.dot(x_ref[...], w2[0:64], preferred_element_type=jnp.float32)
              + jnp.dot(x11_ref[...], w2[64:128], preferred_element_type=jnp.float32)
              + jnp.dot(x12_ref[...], w2[128:192], preferred_element_type=jnp.float32)
              + jnp.dot(x21_ref[...], w2[192:320], preferred_element_type=jnp.float32)
              + jnp.dot(x22, w2[320:448], preferred_element_type=jnp.float32)
              + b2_ref[...])
    o_ref[...] = 1.0 / (1.0 + jnp.exp(-logits))


def kernel(features, edge_index, W1, b1, Wc1, bc1, Wc2, bc2, W2, b2):
    src = edge_index[0]
    dst = edge_index[1]
    z64 = jnp.zeros((_N, 64), jnp.float32)
    z128 = jnp.zeros((_N, 128), jnp.float32)
    b1r = b1.reshape(1, -1)
    bc1r = bc1.reshape(1, -1)
    bc2r = bc2.reshape(1, -1)
    b2r = b2.reshape(1, -1)

    x, h11 = _tc_call(_stage1_body, [features], [W1, b1r, Wc1],
                      [(_N, 64), (_N, 64)])
    p = _sc_aggregate(h11, src, dst, z64, d=64, chunk=1000)
    x11, h12 = _tc_call(_combine_body, [p[0], p[1]], [bc1r, Wc1],
                        [(_N, 64), (_N, 64)])
    p = _sc_aggregate(h12, src, dst, z64, d=64, chunk=1000)
    x12, h21 = _tc_call(_combine2_body, [p[0], p[1], x11],
                        [bc1r, Wc2[0:64], Wc2[64:128]],
                        [(_N, 64), (_N, 128)])
    p = _sc_aggregate(h21, src, dst, z128, d=128, chunk=200)
    x21, h22 = _tc_call(_combine_body, [p[0], p[1]], [bc2r, Wc2],
                        [(_N, 128), (_N, 128)])
    p = _sc_aggregate(h22, src, dst, z128, d=128, chunk=200)
    out = _tc_call(_final_body, [p[0], p[1], x, x11, x12, x21],
                   [bc2r, W2, b2r], [(_N, 32)])
    return out


# speedup vs baseline: 8.9346x; 1.0341x over previous
import functools

import jax
import jax.numpy as jnp
from jax import lax
from jax.experimental import pallas as pl
from jax.experimental.pallas import tpu as pltpu
from jax.experimental.pallas import tpu_sc as plsc

_N = 10000
_E = 320000
_NC = 2
_NS = 16
_EPW = _E // (_NC * _NS)
_RPT = 624
_TAIL = _N - _NS * _RPT
_CH = 400
_NCHUNK = _EPW // _CH


def _sc_aggregate(h_list, src, dst, zeros):
    n_pass = len(h_list)
    mesh = plsc.VectorSubcoreMesh(core_axis_name="c", subcore_axis_name="s",
                                  num_cores=_NC, num_subcores=_NS)

    @functools.partial(
        pl.kernel,
        out_type=jax.ShapeDtypeStruct((_NC, n_pass, _N, 64), jnp.float32),
        mesh=mesh,
        compiler_params=pltpu.CompilerParams(use_tc_tiling_on_sc=False),
        scratch_types=[
            pltpu.VMEM((_CH,), jnp.int32),
            pltpu.VMEM((_CH,), jnp.int32),
            pltpu.VMEM((_CH, 64), jnp.float32),
            pltpu.VMEM((_CH,), jnp.int32),
            pltpu.VMEM((_CH,), jnp.int32),
            pltpu.VMEM((_CH, 64), jnp.float32),
            pltpu.VMEM_SHARED((_N, 64), jnp.float32),
            pltpu.SemaphoreType.DMA,
            pltpu.SemaphoreType.DMA,
            pltpu.SemaphoreType.DMA,
            pltpu.SemaphoreType.DMA,
        ],
    )
    def agg(*refs):
        h_hbms = refs[:n_pass]
        (src_hbm, dst_hbm, z_hbm, out_hbm,
         srcA, dstA, rowsA, srcB, dstB, rowsB,
         acc, gsemA, gsemB, ssemA, ssemB) = refs[n_pass:]
        c = lax.axis_index("c")
        s = lax.axis_index("s")
        ebase = (c * _NS + s) * _EPW

        def load_idx(off, src_v, dst_v):
            pltpu.sync_copy(src_hbm.at[pl.ds(off, _CH)], src_v)
            pltpu.sync_copy(dst_hbm.at[pl.ds(off, _CH)], dst_v)

        for p in range(n_pass):
            h_hbm = h_hbms[p]

            def gather(src_v, rows_v, gsem):
                pltpu.async_copy(h_hbm.at[src_v], rows_v, gsem)

            def gwait(src_v, rows_v, gsem):
                pltpu.make_async_copy(h_hbm.at[src_v], rows_v, gsem).wait()

            def scat(rows_v, dst_v, ssem):
                pltpu.async_copy(rows_v, acc.at[dst_v], ssem, add=True)

            def swait(rows_v, dst_v, ssem):
                pltpu.make_async_copy(rows_v, acc.at[dst_v], ssem).wait()

            pltpu.sync_copy(z_hbm.at[pl.ds(s * _RPT, _RPT)],
                            acc.at[pl.ds(s * _RPT, _RPT)])

            @pl.when(s == 0)
            def _():
                pltpu.sync_copy(z_hbm.at[pl.ds(_NS * _RPT, _TAIL)],
                                acc.at[pl.ds(_NS * _RPT, _TAIL)])

            plsc.subcore_barrier()

            load_idx(ebase, srcA, dstA)
            gather(srcA, rowsA, gsemA)
            load_idx(ebase + _CH, srcB, dstB)
            gather(srcB, rowsB, gsemB)
            gwait(srcA, rowsA, gsemA)
            scat(rowsA, dstA, ssemA)

            def pair(j, carry):
                i = 2 * j + 1
                gwait(srcB, rowsB, gsemB)
                scat(rowsB, dstB, ssemB)
                swait(rowsA, dstA, ssemA)
                load_idx(ebase + (i + 1) * _CH, srcA, dstA)
                gather(srcA, rowsA, gsemA)
                gwait(srcA, rowsA, gsemA)
                scat(rowsA, dstA, ssemA)
                swait(rowsB, dstB, ssemB)

                @pl.when(j < (_NCHUNK - 1) // 2 - 1)
                def _():
                    load_idx(ebase + (i + 2) * _CH, srcB, dstB)
                    gather(srcB, rowsB, gsemB)

                return carry

            lax.fori_loop(0, (_NCHUNK - 1) // 2, pair, 0)
            swait(rowsA, dstA, ssemA)
            plsc.subcore_barrier()
            pltpu.sync_copy(acc.at[pl.ds(s * _RPT, _RPT)],
                            out_hbm.at[c, p, pl.ds(s * _RPT, _RPT)])

            @pl.when(s == 0)
            def _():
                pltpu.sync_copy(acc.at[pl.ds(_NS * _RPT, _TAIL)],
                                out_hbm.at[c, p, pl.ds(_NS * _RPT, _TAIL)])

            if p + 1 < n_pass:
                plsc.subcore_barrier()

    return agg(*h_list, src, dst, zeros)


_BLK = 1000


def _row_specs(n_in, shapes_in, shapes_out):
    in_specs = [pl.BlockSpec((_BLK,) + s[1:], lambda i, *, _nd=len(s): (i,) + (0,) * (_nd - 1))
                for s in shapes_in[:n_in]]
    in_specs += [pl.BlockSpec(s, lambda i, *, _nd=len(s): (0,) * _nd)
                 for s in shapes_in[n_in:]]
    out_specs = [pl.BlockSpec((_BLK,) + s[1:], lambda i, *, _nd=len(s): (i,) + (0,) * (_nd - 1))
                 for s in shapes_out]
    return in_specs, out_specs


def _tc_call(body, row_args, full_args, out_shapes):
    shapes_in = [a.shape for a in row_args] + [a.shape for a in full_args]
    in_specs, out_specs = _row_specs(len(row_args), shapes_in, out_shapes)
    return pl.pallas_call(
        body,
        grid=(_N // _BLK,),
        in_specs=in_specs,
        out_specs=out_specs if len(out_shapes) > 1 else out_specs[0],
        out_shape=[jax.ShapeDtypeStruct(s, jnp.float32) for s in out_shapes]
        if len(out_shapes) > 1 else jax.ShapeDtypeStruct(out_shapes[0], jnp.float32),
    )(*row_args, *full_args)


def _dot(a, b):
    return jnp.dot(a, b, preferred_element_type=jnp.float32)


def _stage1_body(f_ref, w1_ref, b1_ref, wc1_ref, x_ref, h_ref):
    x = jnp.maximum(_dot(f_ref[...], w1_ref[...]) + b1_ref[...], 0.0)
    x_ref[...] = x
    h_ref[...] = _dot(x, wc1_ref[...])


def _combine_body(p0_ref, p1_ref, b_ref, w_ref, x_ref, h_ref):
    x = p0_ref[...] + p1_ref[...] + b_ref[...]
    x_ref[...] = x
    h_ref[...] = _dot(x, w_ref[...])


def _combine2_body(p0_ref, p1_ref, xa_ref, b_ref, w_ref,
                   x_ref, hlo_ref, hhi_ref):
    x = p0_ref[...] + p1_ref[...] + b_ref[...]
    x_ref[...] = x
    w = w_ref[...]
    hlo_ref[...] = _dot(xa_ref[...], w[0:64, 0:64]) + _dot(x, w[64:128, 0:64])
    hhi_ref[...] = _dot(xa_ref[...], w[0:64, 64:128]) + _dot(x, w[64:128, 64:128])


def _combine4_body(p00_ref, p10_ref, p01_ref, p11_ref, blo_ref, bhi_ref,
                   w_ref, xlo_ref, xhi_ref, hlo_ref, hhi_ref):
    xlo = p00_ref[...] + p10_ref[...] + blo_ref[...]
    xhi = p01_ref[...] + p11_ref[...] + bhi_ref[...]
    xlo_ref[...] = xlo
    xhi_ref[...] = xhi
    w = w_ref[...]
    hlo_ref[...] = _dot(xlo, w[0:64, 0:64]) + _dot(xhi, w[64:128, 0:64])
    hhi_ref[...] = _dot(xlo, w[0:64, 64:128]) + _dot(xhi, w[64:128, 64:128])


def _final_body(p00_ref, p10_ref, p01_ref, p11_ref, x_ref, x11_ref, x12_ref,
                x21lo_ref, x21hi_ref, blo_ref, bhi_ref, w2_ref, b2_ref, o_ref):
    x22lo = p00_ref[...] + p10_ref[...] + blo_ref[...]
    x22hi = p01_ref[...] + p11_ref[...] + bhi_ref[...]
    w2 = w2_ref[...]
    logits = (_dot(x_ref[...], w2[0:64])
              + _dot(x11_ref[...], w2[64:128])
              + _dot(x12_ref[...], w2[128:192])
              + _dot(x21lo_ref[...], w2[192:256])
              + _dot(x21hi_ref[...], w2[256:320])
              + _dot(x22lo, w2[320:384])
              + _dot(x22hi, w2[384:448])
              + b2_ref[...])
    o_ref[...] = 1.0 / (1.0 + jnp.exp(-logits))


def kernel(features, edge_index, W1, b1, Wc1, bc1, Wc2, bc2, W2, b2):
    src = edge_index[0]
    dst = edge_index[1]
    z64 = jnp.zeros((_N, 64), jnp.float32)
    b1r = b1.reshape(1, -1)
    bc1r = bc1.reshape(1, -1)
    bc2lo = bc2[0:64].reshape(1, -1)
    bc2hi = bc2[64:128].reshape(1, -1)
    b2r = b2.reshape(1, -1)

    x, h11 = _tc_call(_stage1_body, [features], [W1, b1r, Wc1],
                      [(_N, 64), (_N, 64)])
    p = _sc_aggregate([h11], src, dst, z64)
    x11, h12 = _tc_call(_combine_body, [p[0, 0], p[1, 0]], [bc1r, Wc1],
                        [(_N, 64), (_N, 64)])
    p = _sc_aggregate([h12], src, dst, z64)
    x12, h21lo, h21hi = _tc_call(_combine2_body, [p[0, 0], p[1, 0], x11],
                                 [bc1r, Wc2],
                                 [(_N, 64), (_N, 64), (_N, 64)])
    p = _sc_aggregate([h21lo, h21hi], src, dst, z64)
    x21lo, x21hi, h22lo, h22hi = _tc_call(
        _combine4_body, [p[0, 0], p[1, 0], p[0, 1], p[1, 1]],
        [bc2lo, bc2hi, Wc2],
        [(_N, 64), (_N, 64), (_N, 64), (_N, 64)])
    p = _sc_aggregate([h22lo, h22hi], src, dst, z64)
    out = _tc_call(_final_body,
                   [p[0, 0], p[1, 0], p[0, 1], p[1, 1], x, x11, x12,
                    x21lo, x21hi],
                   [bc2lo, bc2hi, W2, b2r], [(_N, 32)])
    return out
